# Initial kernel scaffold; baseline (speedup 1.0000x reference)
import jax
import jax.numpy as jnp
from jax.experimental import pallas as pl


def kernel(tokens, row_lengths, max_sequence_length, embeddings):
    raise NotImplementedError("write your pallas kernel here")


# speedup vs baseline: 2.1791x; 2.1791x over previous
import functools

import jax
import jax.numpy as jnp
from jax import lax
from jax.experimental import pallas as pl
from jax.experimental.pallas import tpu as pltpu
from jax.experimental.pallas import tpu_sc as plsc

_B = 4096
_S = 200
_D = 64
_CH = 40
_NCH = _S // _CH
_NW = 32
_RPW = _B // _NW
_EPS = 1e-08


def _feeder_body(tokens_h, lens_h, emb_h, x_h, dec_h, ts_h,
                 idx_v, rows_v, stair_v, zero_v, lens_v, ts_v, gsem):
    cid = lax.axis_index("c")
    sid = lax.axis_index("s")
    wid = cid * 16 + sid
    base = wid * _RPW

    zv = jnp.zeros((16,), jnp.float32)

    def init_stair(p, carry):
        val = jnp.where(p < _CH, jnp.float32(1.0 + _EPS), jnp.float32(_EPS))
        v = jnp.full((16,), val, jnp.float32)
        for k in range(_D // 16):
            stair_v[p, pl.ds(k * 16, 16)] = v
        return carry

    lax.fori_loop(0, 2 * _CH, init_stair, 0)

    def init_zero(p, carry):
        for k in range(_D // 16):
            zero_v[p, pl.ds(k * 16, 16)] = zv
        return carry

    lax.fori_loop(0, _CH, init_zero, 0)

    pltpu.sync_copy(lens_h, lens_v.at[pl.ds(0, _B)])

    @pl.when(wid == 0)
    def _():
        def mx(i, acc):
            return jnp.maximum(acc, lens_v[pl.ds(i * 16, 16)])

        m = lax.fori_loop(0, _B // 16, mx, jnp.zeros((16,), jnp.int32))
        m = jnp.minimum(m, _S)
        mm = m[0]
        for j in range(1, 16):
            mm = jnp.maximum(mm, m[j])
        ts_v[:] = jnp.full((16,), mm, jnp.int32)
        pltpu.sync_copy(ts_v, ts_h)

    def row_body(i, carry):
        row = base + i
        ln = jnp.minimum(lens_v[pl.ds(row, 16)][0], _S)
        nfull = ln // _CH
        rem = ln % _CH
        nch = nfull + jnp.where(rem > 0, 1, 0)

        pltpu.sync_copy(tokens_h.at[row], idx_v)

        handles = []
        for c in range(_NCH):
            handles.append(
                pltpu.async_copy(emb_h.at[idx_v.at[c]], rows_v.at[c], gsem))
        for h in handles:
            h.wait()

        @pl.when(rem > 0)
        def _():
            def zb(p, carry2):
                for k in range(_D // 16):
                    rows_v[nfull, p, pl.ds(k * 16, 16)] = zv
                return carry2

            lax.fori_loop(rem, _CH, zb, 0)

        for c in range(_NCH):
            valid = c < nch

            @pl.when(valid)
            def _():
                pltpu.sync_copy(rows_v.at[c], x_h.at[row, c])

            @pl.when(jnp.logical_not(valid))
            def _():
                pltpu.sync_copy(zero_v, x_h.at[row, c])

            r_c = jnp.clip(ln - c * _CH, 0, _CH)
            pltpu.sync_copy(stair_v.at[pl.ds(_CH - r_c, _CH)],
                            dec_h.at[row, c])
        return carry

    lax.fori_loop(0, _RPW, row_body, 0)


def kernel(tokens, row_lengths, max_sequence_length, embeddings):
    del max_sequence_length
    tokens_r = tokens.reshape(_B, _NCH, _CH)

    feeder = pl.kernel(
        _feeder_body,
        out_type=(
            jax.ShapeDtypeStruct((_B, _NCH, _CH, _D), jnp.float32),
            jax.ShapeDtypeStruct((_B, _NCH, _CH, _D), jnp.float32),
            jax.ShapeDtypeStruct((16,), jnp.int32),
        ),
        mesh=plsc.VectorSubcoreMesh(core_axis_name="c", subcore_axis_name="s"),
        compiler_params=pltpu.CompilerParams(use_tc_tiling_on_sc=False),
        scratch_types=[
            pltpu.VMEM((_NCH, _CH), jnp.int32),
            pltpu.VMEM((_NCH, _CH, _D), jnp.float32),
            pltpu.VMEM((2 * _CH, _D), jnp.float32),
            pltpu.VMEM((_CH, _D), jnp.float32),
            pltpu.VMEM((_B + 16,), jnp.int32),
            pltpu.VMEM((16,), jnp.int32),
            pltpu.SemaphoreType.DMA,
        ],
    )
    x, dec, ts = feeder(tokens_r, row_lengths, embeddings)
    return (x.reshape(_B, _S, _D), dec.reshape(_B, _S, _D), ts[0])


# speedup vs baseline: 2.5868x; 1.1871x over previous
import functools

import jax
import jax.numpy as jnp
from jax import lax
from jax.experimental import pallas as pl
from jax.experimental.pallas import tpu as pltpu
from jax.experimental.pallas import tpu_sc as plsc

_B = 4096
_S = 200
_D = 64
_CH = 40
_NCH = _S // _CH
_NW = 32
_RPW = _B // _NW
_NPAIR = _RPW // 2
_EPS = 1e-08


def _feeder_body(tokens_h, lens_h, emb_h, x_h, dec_h, ts_h,
                 idx_v, rows_v, stair_v, zero_v, lens_v, ts_v,
                 gsem, xsem, dsem):
    cid = lax.axis_index("c")
    sid = lax.axis_index("s")
    wid = cid * 16 + sid
    base = wid * _RPW

    zv = jnp.zeros((16,), jnp.float32)

    def init_stair(p, carry):
        val = jnp.where(p < _CH, jnp.float32(1.0 + _EPS), jnp.float32(_EPS))
        v = jnp.full((16,), val, jnp.float32)
        for k in range(_D // 16):
            stair_v[p, pl.ds(k * 16, 16)] = v
        return carry

    lax.fori_loop(0, 2 * _CH, init_stair, 0)

    def init_zero(p, carry):
        for k in range(_D // 16):
            zero_v[p, pl.ds(k * 16, 16)] = zv
        return carry

    lax.fori_loop(0, _CH, init_zero, 0)

    pltpu.sync_copy(lens_h, lens_v.at[pl.ds(0, _B)])

    @pl.when(wid == 0)
    def _():
        def mx(i, acc):
            return jnp.maximum(acc, lens_v[pl.ds(i * 16, 16)])

        m = lax.fori_loop(0, _B // 16, mx, jnp.zeros((16,), jnp.int32))
        m = jnp.minimum(m, _S)
        mm = m[0]
        for j in range(1, 16):
            mm = jnp.maximum(mm, m[j])
        ts_v[:] = jnp.full((16,), mm, jnp.int32)
        pltpu.sync_copy(ts_v, ts_h)

    def drain_outputs(row):
        for c in range(_NCH):
            for _ in range(2):
                pltpu.make_async_copy(zero_v, x_h.at[row, c], xsem).wait()
                pltpu.make_async_copy(stair_v.at[pl.ds(0, _CH)],
                                      dec_h.at[row, c], dsem).wait()

    def do_row(row, buf):
        ln = jnp.minimum(lens_v[pl.ds(row, 16)][0], _S)
        nfull = ln // _CH
        rem = ln % _CH
        nch = nfull + jnp.where(rem > 0, 1, 0)

        for c in range(_NCH):
            @pl.when(c < nch)
            def _():
                pltpu.async_copy(emb_h.at[idx_v.at[buf, c]],
                                 rows_v.at[buf, c], gsem)

        for c in range(_NCH):
            r_c = jnp.clip(ln - c * _CH, 0, _CH)
            pltpu.async_copy(stair_v.at[pl.ds(_CH - r_c, _CH)],
                             dec_h.at[row, c], dsem)
        return ln, nfull, rem, nch

    def finish_row(row, buf, ln, nfull, rem, nch):
        for c in range(_NCH):
            @pl.when(c < nch)
            def _():
                pltpu.make_async_copy(emb_h.at[idx_v.at[buf, c]],
                                      rows_v.at[buf, c], gsem).wait()

        @pl.when(rem > 0)
        def _():
            def zb(p, carry2):
                for k in range(_D // 16):
                    rows_v[buf, nfull, p, pl.ds(k * 16, 16)] = zv
                return carry2

            lax.fori_loop(rem, _CH, zb, 0)

        for c in range(_NCH):
            @pl.when(c < nch)
            def _():
                pltpu.async_copy(rows_v.at[buf, c], x_h.at[row, c], xsem)

            @pl.when(c >= nch)
            def _():
                pltpu.async_copy(zero_v, x_h.at[row, c], xsem)

    def pair_body(j, carry):
        a = base + 2 * j
        b = a + 1
        pa = (j & 1) * 2
        pb = pa + 1

        @pl.when(j > 0)
        def _():
            drain_outputs(a)

        pltpu.sync_copy(tokens_h.at[pl.ds(a, 2)],
                        idx_v.at[pl.ds(pa, 2)])

        sa = do_row(a, pa)
        sb = do_row(b, pb)
        finish_row(a, pa, *sa)
        finish_row(b, pb, *sb)
        return carry

    lax.fori_loop(0, _NPAIR, pair_body, 0)
    drain_outputs(base)


def kernel(tokens, row_lengths, max_sequence_length, embeddings):
    del max_sequence_length
    tokens_r = tokens.reshape(_B, _NCH, _CH)

    feeder = pl.kernel(
        _feeder_body,
        out_type=(
            jax.ShapeDtypeStruct((_B, _NCH, _CH, _D), jnp.float32),
            jax.ShapeDtypeStruct((_B, _NCH, _CH, _D), jnp.float32),
            jax.ShapeDtypeStruct((16,), jnp.int32),
        ),
        mesh=plsc.VectorSubcoreMesh(core_axis_name="c", subcore_axis_name="s"),
        compiler_params=pltpu.CompilerParams(use_tc_tiling_on_sc=False),
        scratch_types=[
            pltpu.VMEM((4, _NCH, _CH), jnp.int32),
            pltpu.VMEM((4, _NCH, _CH, _D), jnp.float32),
            pltpu.VMEM((2 * _CH, _D), jnp.float32),
            pltpu.VMEM((_CH, _D), jnp.float32),
            pltpu.VMEM((_B + 16,), jnp.int32),
            pltpu.VMEM((16,), jnp.int32),
            pltpu.SemaphoreType.DMA,
            pltpu.SemaphoreType.DMA,
            pltpu.SemaphoreType.DMA,
        ],
    )
    x, dec, ts = feeder(tokens_r, row_lengths, embeddings)
    return (x.reshape(_B, _S, _D), dec.reshape(_B, _S, _D), ts[0])
